# Initial kernel scaffold; baseline (speedup 1.0000x reference)
import jax
import jax.numpy as jnp
from jax.experimental import pallas as pl


def kernel(s_j, v_j, r_ij, nbrs, W1, b1, W2, b2, Wd, bd):
    raise NotImplementedError("write your pallas kernel here")


# speedup vs baseline: 2.7683x; 2.7683x over previous
import functools
import math

import jax
import jax.numpy as jnp
from jax import lax
from jax.experimental import pallas as pl
from jax.experimental.pallas import tpu as pltpu
from jax.experimental.pallas import tpu_sc as plsc

N_RBF = 20
CUTOFF = 5.0
FEAT = 128
HALF = 64
LANES = 16

NC = 2
NS = 16
C_E = 48
N_PH = 4

D_ROW = 4 * HALF
D_WS = 4 * HALF
D_OUT = 2 * HALF
N_LO = 8000
N_ACC = 8064


def _phi_body(s_ref, w1_ref, b1_ref, w2_ref, b2_ref, out_ref):
    h = jnp.dot(s_ref[...], w1_ref[...], preferred_element_type=jnp.float32)
    h = h + b1_ref[...]
    h = h * jax.nn.sigmoid(h)
    out = jnp.dot(h, w2_ref[...], preferred_element_type=jnp.float32)
    out_ref[...] = out + b2_ref[...]


def _phi_nodes(s_j, W1, b1, W2, b2):
    n = s_j.shape[0]
    bn = 2000
    assert n % bn == 0
    return pl.pallas_call(
        _phi_body,
        grid=(n // bn,),
        in_specs=[
            pl.BlockSpec((bn, FEAT), lambda i: (i, 0)),
            pl.BlockSpec((FEAT, FEAT), lambda i: (0, 0)),
            pl.BlockSpec((1, FEAT), lambda i: (0, 0)),
            pl.BlockSpec((FEAT, 3 * FEAT), lambda i: (0, 0)),
            pl.BlockSpec((1, 3 * FEAT), lambda i: (0, 0)),
        ],
        out_specs=pl.BlockSpec((bn, 3 * FEAT), lambda i: (i, 0)),
        out_shape=jax.ShapeDtypeStruct((n, 3 * FEAT), jnp.float32),
    )(s_j, W1, b1[None, :], W2, b2[None, :])


def _ws_body(r_ref, wd_ref, bd_ref, out_ref):
    r = r_ref[...]
    d2 = jnp.sum(r * r, axis=1, keepdims=True) + 3e-8
    d = jnp.sqrt(d2)
    unit = r / d
    n = lax.broadcasted_iota(jnp.int32, (1, N_RBF), 1).astype(jnp.float32) + 1.0
    rbf = jnp.sin(d * (n * (math.pi / CUTOFF))) / d
    ws = jnp.dot(rbf, wd_ref[...], preferred_element_type=jnp.float32)
    ws = ws + bd_ref[...]
    env = jnp.where(d < CUTOFF, 0.5 * (jnp.cos(d * (math.pi / CUTOFF)) + 1.0), 0.0)
    ws = ws * env
    w0 = ws[:, :FEAT]
    w1 = ws[:, FEAT:2 * FEAT]
    w2 = ws[:, 2 * FEAT:]
    w2u = [w2 * unit[:, a:a + 1] for a in range(3)]
    pad = jnp.zeros((r.shape[0], HALF), jnp.float32)
    for h in range(2):
        sl = slice(HALF * h, HALF * (h + 1))
        out_ref[h] = jnp.concatenate(
            [w1[:, sl], w0[:, sl], w2u[0][:, sl], pad], axis=1)
        out_ref[2 + h] = jnp.concatenate(
            [w0[:, sl], w2u[1][:, sl], w2u[2][:, sl], pad], axis=1)


def _edge_ws(r_pad, Wd, bd):
    e_pad = r_pad.shape[0]
    be = 768
    assert e_pad % be == 0
    return pl.pallas_call(
        _ws_body,
        grid=(e_pad // be,),
        in_specs=[
            pl.BlockSpec((be, 3), lambda i: (i, 0)),
            pl.BlockSpec((N_RBF, 3 * FEAT), lambda i: (0, 0)),
            pl.BlockSpec((1, 3 * FEAT), lambda i: (0, 0)),
        ],
        out_specs=pl.BlockSpec((4, be, D_WS), lambda i: (0, i, 0)),
        out_shape=jax.ShapeDtypeStruct((4, e_pad, D_WS), jnp.float32),
    )(r_pad, Wd, bd[None, :])


def _edge_compute_a(e, rows_v, ws_v, msg_v):
    for k in range(HALF // LANES):
        s0 = pl.ds(k * LANES, LANES)
        s1 = pl.ds(HALF + k * LANES, LANES)
        s2 = pl.ds(2 * HALF + k * LANES, LANES)
        s3 = pl.ds(3 * HALF + k * LANES, LANES)
        msg_v[e, s0] = rows_v[e, s0] * ws_v[e, s0]
        t0 = rows_v[e, s1] * ws_v[e, s1]
        msg_v[e, s1] = t0 * rows_v[e, s3] + rows_v[e, s2] * ws_v[e, s2]


def _edge_compute_b(e, rows_v, ws_v, msg_v):
    for k in range(HALF // LANES):
        s0 = pl.ds(k * LANES, LANES)
        s1 = pl.ds(HALF + k * LANES, LANES)
        s2 = pl.ds(2 * HALF + k * LANES, LANES)
        s3 = pl.ds(3 * HALF + k * LANES, LANES)
        t0 = rows_v[e, s0] * ws_v[e, s0]
        p2 = rows_v[e, s1]
        msg_v[e, s0] = t0 * rows_v[e, s2] + p2 * ws_v[e, s1]
        msg_v[e, s1] = t0 * rows_v[e, s3] + p2 * ws_v[e, s2]


def _make_sc_kernel(n, e_pad):
    eps = e_pad // NS
    nch = eps // C_E
    npt = N_ACC // NS
    assert eps % C_E == 0 and npt % 8 == 0

    mesh = plsc.VectorSubcoreMesh(
        core_axis_name="c", subcore_axis_name="s",
        num_cores=NC, num_subcores=NS)

    @functools.partial(
        pl.kernel,
        out_type=jax.ShapeDtypeStruct((2 * N_PH * N_ACC, D_OUT), jnp.float32),
        mesh=mesh,
        scratch_types=[
            pltpu.VMEM((C_E,), jnp.int32),
            pltpu.VMEM((C_E,), jnp.int32),
            pltpu.VMEM((C_E, D_ROW), jnp.float32),
            pltpu.VMEM((C_E, D_WS), jnp.float32),
            pltpu.VMEM((C_E, D_OUT), jnp.float32),
            pltpu.VMEM_SHARED((N_ACC, D_OUT), jnp.float32),
            pltpu.SemaphoreType.DMA,
        ],
    )
    def sc_kernel(table_hbm, srcs_hbm, dst_hbm, ws_hbm, zeros_hbm, out_hbm,
                  src_v, dst_v, rows_v, ws_v, msg_v, acc, sem):
        c = lax.axis_index("c")
        s = lax.axis_index("s")
        ebase = s * eps
        nbase = s * npt

        def run_phase(p, edge_compute):
            ipair = (p // 2) * 2 + c
            irange = p % 2
            iout = p * 2 + c

            pltpu.sync_copy(zeros_hbm, msg_v)
            done = 0
            while done < npt:
                rows = min(C_E, npt - done)
                pltpu.sync_copy(msg_v.at[pl.ds(0, rows)],
                                acc.at[pl.ds(nbase + done, rows)])
                done += rows
            plsc.subcore_barrier()

            def chunk_body(j, _):
                e0 = ebase + j * C_E
                pltpu.sync_copy(srcs_hbm.at[pl.ds(ipair * e_pad + e0, C_E)], src_v)
                pltpu.sync_copy(dst_hbm.at[pl.ds(irange * e_pad + e0, C_E)], dst_v)
                pltpu.async_copy(table_hbm.at[src_v], rows_v, sem).wait()
                pltpu.sync_copy(ws_hbm.at[pl.ds(ipair * e_pad + e0, C_E)], ws_v)

                for e in range(C_E):
                    edge_compute(e, rows_v, ws_v, msg_v)

                pltpu.sync_copy(msg_v, acc.at[dst_v], add=True)
                return 0
            lax.fori_loop(0, nch, chunk_body, 0)
            plsc.subcore_barrier()

            done = 0
            while done < npt:
                rows = min(C_E, npt - done)
                pltpu.sync_copy(acc.at[pl.ds(nbase + done, rows)],
                                msg_v.at[pl.ds(0, rows)])
                pltpu.sync_copy(msg_v.at[pl.ds(0, rows)],
                                out_hbm.at[pl.ds(iout * N_ACC + nbase + done, rows)])
                done += rows
            plsc.subcore_barrier()

        def phases_a(p, _):
            run_phase(p, _edge_compute_a)
            return 0

        def phases_b(p, _):
            run_phase(p, _edge_compute_b)
            return 0

        lax.fori_loop(0, 2, phases_a, 0)
        lax.fori_loop(2, 4, phases_b, 0)

    return sc_kernel


def kernel(s_j, v_j, r_ij, nbrs, W1, b1, W2, b2, Wd, bd):
    n = s_j.shape[0]
    e = r_ij.shape[0]
    e_pad = ((e + NS * C_E - 1) // (NS * C_E)) * (NS * C_E)
    n_hi = n - N_LO

    pad_n = e_pad - e
    r_pad = jnp.concatenate(
        [r_ij, jnp.tile(jnp.array([[2.0 * CUTOFF, 0.0, 0.0]], jnp.float32),
                        (pad_n, 1))], axis=0)
    nbrs_pad = jnp.concatenate(
        [nbrs.astype(jnp.int32), jnp.zeros((pad_n, 2), jnp.int32)], axis=0)
    dst = nbrs_pad[:, 0]
    src = nbrs_pad[:, 1]
    srcs_all = (src[None, :]
                + (jnp.arange(4, dtype=jnp.int32) * n)[:, None]).reshape(-1)
    dst_all = jnp.concatenate([
        jnp.where(dst < N_LO, dst, N_LO),
        jnp.where(dst >= N_LO, dst - N_LO, n_hi),
    ])

    phi = _phi_nodes(s_j, W1, b1, W2, b2)
    ws_all = _edge_ws(r_pad, Wd, bd)

    phi0 = phi[:, :FEAT]
    phi1 = phi[:, FEAT:2 * FEAT]
    phi2 = phi[:, 2 * FEAT:]
    va = [v_j[:, :, a] for a in range(3)]
    blocks = []
    for h in range(2):
        sl = slice(HALF * h, HALF * (h + 1))
        blocks.append(jnp.concatenate(
            [phi1[:, sl], phi0[:, sl], phi2[:, sl], va[0][:, sl]], axis=1))
    for h in range(2):
        sl = slice(HALF * h, HALF * (h + 1))
        blocks.append(jnp.concatenate(
            [phi0[:, sl], phi2[:, sl], va[1][:, sl], va[2][:, sl]], axis=1))
    table_all = jnp.concatenate(blocks, axis=0)

    sc = _make_sc_kernel(n, e_pad)
    zeros = jnp.zeros((C_E, D_OUT), jnp.float32)
    out_all = sc(table_all, srcs_all.reshape(-1), dst_all, ws_all.reshape(-1, D_WS),
                 zeros).reshape(2 * N_PH, N_ACC, D_OUT)

    def full(pair, c):
        lo = out_all[(pair * 2 + 0) * 2 + c, :N_LO]
        hi = out_all[(pair * 2 + 1) * 2 + c, :n_hi]
        return jnp.concatenate([lo, hi], axis=0)

    pa = [full(0, c) for c in range(2)]
    pb = [full(1, c) for c in range(2)]
    delta_s = jnp.concatenate([pa[0][:, :HALF], pa[1][:, :HALF]], axis=1)
    a0 = jnp.concatenate([pa[0][:, HALF:], pa[1][:, HALF:]], axis=1)
    a1 = jnp.concatenate([pb[0][:, :HALF], pb[1][:, :HALF]], axis=1)
    a2 = jnp.concatenate([pb[0][:, HALF:], pb[1][:, HALF:]], axis=1)
    delta_v = jnp.stack([a0, a1, a2], axis=2)
    return (delta_s, delta_v)
